# Initial kernel scaffold; baseline (speedup 1.0000x reference)
import functools

import jax
import jax.numpy as jnp
from jax import lax
from jax.experimental import pallas as pl
from jax.experimental.pallas import tpu as pltpu
from jax.experimental.pallas import tpu_sc as plsc

NUM_CORES = 2
NUM_SUBCORES = 16
NUM_WORKERS = NUM_CORES * NUM_SUBCORES

BATCH = 16384
HIST = 50
EMBED_DIM = 64
TOTAL = BATCH * HIST
ROWS_PER_WORKER = TOTAL // NUM_WORKERS
CHUNK = 512
N_CHUNKS = ROWS_PER_WORKER // CHUNK


def _emb_kernel(idx_hbm, table_hbm, out_hbm, idx_v, rows_v, sem):
    wid = lax.axis_index("s") * NUM_CORES + lax.axis_index("c")
    wbase = wid * ROWS_PER_WORKER

    @pl.loop(0, N_CHUNKS)
    def _chunk(c):
        base = wbase + c * CHUNK
        pltpu.sync_copy(idx_hbm.at[pl.ds(base, CHUNK)], idx_v)
        pltpu.async_copy(table_hbm.at[idx_v], rows_v, sem).wait()
        pltpu.sync_copy(rows_v, out_hbm.at[pl.ds(base, CHUNK)])


@jax.jit
def _embedding_lookup(idx_flat, w):
    mesh = plsc.VectorSubcoreMesh(core_axis_name="c", subcore_axis_name="s")
    run = pl.kernel(
        _emb_kernel,
        out_type=jax.ShapeDtypeStruct((TOTAL, EMBED_DIM), jnp.float32),
        mesh=mesh,
        scratch_types=[
            pltpu.VMEM((CHUNK,), jnp.int32),
            pltpu.VMEM((CHUNK, EMBED_DIM), jnp.float32),
            pltpu.SemaphoreType.DMA,
        ],
    )
    return run(idx_flat, w)


def kernel(x, w):
    idx_flat = x.reshape(-1).astype(jnp.int32)
    out = _embedding_lookup(idx_flat, w)
    return out.reshape(x.shape + (EMBED_DIM,))


# speedup vs baseline: 1.7960x; 1.7960x over previous
import functools

import jax
import jax.numpy as jnp
from jax import lax
from jax.experimental import pallas as pl
from jax.experimental.pallas import tpu as pltpu
from jax.experimental.pallas import tpu_sc as plsc

NUM_CORES = 2
NUM_SUBCORES = 16
NUM_WORKERS = NUM_CORES * NUM_SUBCORES

BATCH = 16384
HIST = 50
EMBED_DIM = 64
TOTAL = BATCH * HIST
ROWS_PER_WORKER = TOTAL // NUM_WORKERS
CHUNK = 512
N_CHUNKS = ROWS_PER_WORKER // CHUNK


def _emb_kernel(idx_hbm, table_hbm, out_hbm, idx_v, rows_v, sem):
    wid = lax.axis_index("s") * NUM_CORES + lax.axis_index("c")
    wbase = wid * ROWS_PER_WORKER

    @pl.loop(0, N_CHUNKS)
    def _chunk(c):
        base = wbase + c * CHUNK
        pltpu.sync_copy(idx_hbm.at[pl.ds(base, CHUNK)], idx_v)
        pltpu.async_copy(table_hbm.at[idx_v], rows_v, sem).wait()
        pltpu.sync_copy(rows_v, out_hbm.at[pl.ds(base, CHUNK)])


@jax.jit
def _embedding_lookup(idx_flat, w):
    mesh = plsc.VectorSubcoreMesh(core_axis_name="c", subcore_axis_name="s")
    run = pl.kernel(
        _emb_kernel,
        out_type=jax.ShapeDtypeStruct((TOTAL, EMBED_DIM), jnp.float32),
        mesh=mesh,
        scratch_types=[
            pltpu.VMEM((CHUNK,), jnp.int32),
            pltpu.VMEM((CHUNK, EMBED_DIM), jnp.float32),
            pltpu.SemaphoreType.DMA,
        ],
        compiler_params=pltpu.CompilerParams(use_tc_tiling_on_sc=False),
    )
    return run(idx_flat, w)


def kernel(x, w):
    idx_flat = x.reshape(-1).astype(jnp.int32)
    out = _embedding_lookup(idx_flat, w)
    return out.reshape(x.shape + (EMBED_DIM,))


# speedup vs baseline: 1.8748x; 1.0439x over previous
import jax
import jax.numpy as jnp
from jax import lax
from jax.experimental import pallas as pl
from jax.experimental.pallas import tpu as pltpu
from jax.experimental.pallas import tpu_sc as plsc

NUM_CORES = 2
NUM_SUBCORES = 16
NUM_WORKERS = NUM_CORES * NUM_SUBCORES

BATCH = 16384
HIST = 50
EMBED_DIM = 64
TOTAL = BATCH * HIST
ROWS_PER_WORKER = TOTAL // NUM_WORKERS
CHUNK = 256
NBUF = 4
N_CHUNKS = ROWS_PER_WORKER // CHUNK


def _emb_kernel(idx_hbm, table_hbm, out_hbm, idx_all, *scratch):
    rows = scratch[:NBUF]
    gsem = scratch[NBUF:2 * NBUF]
    wsem = scratch[2 * NBUF:3 * NBUF]

    wid = lax.axis_index("s") * NUM_CORES + lax.axis_index("c")
    wbase = wid * ROWS_PER_WORKER

    pltpu.sync_copy(idx_hbm.at[pl.ds(wbase, ROWS_PER_WORKER)], idx_all)

    def idx_slice(c):
        return idx_all.at[pl.ds(c * CHUNK, CHUNK)]

    def out_slice(c):
        return out_hbm.at[pl.ds(wbase + c * CHUNK, CHUNK)]

    for b in range(NBUF):
        pltpu.async_copy(table_hbm.at[idx_slice(b)], rows[b], gsem[b])

    @pl.loop(0, N_CHUNKS, step=NBUF)
    def _round(c0):
        for b in range(NBUF):
            c = c0 + b
            pltpu.make_async_copy(table_hbm.at[idx_slice(c)], rows[b],
                                  gsem[b]).wait()
            pltpu.async_copy(rows[b], out_slice(c), wsem[b])
            nxt = c + NBUF

            @pl.when(nxt < N_CHUNKS)
            def _refill():
                pltpu.make_async_copy(rows[b], out_slice(c), wsem[b]).wait()
                pltpu.async_copy(table_hbm.at[idx_slice(nxt)], rows[b],
                                 gsem[b])

    for b in range(NBUF):
        c_last = N_CHUNKS - NBUF + b
        pltpu.make_async_copy(rows[b], out_slice(c_last), wsem[b]).wait()


@jax.jit
def _embedding_lookup(idx_flat, w):
    mesh = plsc.VectorSubcoreMesh(core_axis_name="c", subcore_axis_name="s")
    run = pl.kernel(
        _emb_kernel,
        out_type=jax.ShapeDtypeStruct((TOTAL, EMBED_DIM), jnp.float32),
        mesh=mesh,
        scratch_types=(
            [pltpu.VMEM((ROWS_PER_WORKER,), jnp.int32)]
            + [pltpu.VMEM((CHUNK, EMBED_DIM), jnp.float32) for _ in range(NBUF)]
            + [pltpu.SemaphoreType.DMA for _ in range(2 * NBUF)]
        ),
        compiler_params=pltpu.CompilerParams(use_tc_tiling_on_sc=False),
    )
    return run(idx_flat, w)


def kernel(x, w):
    idx_flat = x.reshape(-1).astype(jnp.int32)
    out = _embedding_lookup(idx_flat, w)
    return out.reshape(x.shape + (EMBED_DIM,))
